# baseline (device time: 35367 ns/iter reference)
import jax
import jax.numpy as jnp
from jax import lax
from jax.experimental import pallas as pl
from jax.experimental.pallas import tpu as pltpu

N_DEV = 4
B_SH = 64
B = N_DEV * B_SH
D = 512
H_SH = 1024
W = 2
RW = B_SH // W
HW = N_DEV * RW // 2
N_RDMA = 28

F32 = jnp.float32
BF16 = jnp.bfloat16


def kernel(x, Win0, Wout0, Win1, Wout1, Win2, Wout2):
    def body(x_ref, win0_ref, wout0_ref, win1_ref, wout1_ref, win2_ref,
             wout2_ref, out_ref, partb_ref, xb_ref, agb_ref, sab_ref,
             arb_ref, rsb_ref, winb_ref, woutb_ref, send_sems, recv_sems):
        my = lax.axis_index("i")
        y_p = my ^ 1
        x_p = 3 - my
        d_p = (3 - my) ^ 1
        peers = (y_p, x_p, d_p)

        barrier_sem = pltpu.get_barrier_semaphore()
        for nbr in (y_p, x_p):
            pl.semaphore_signal(
                barrier_sem, inc=1,
                device_id=(nbr,), device_id_type=pl.DeviceIdType.MESH,
            )
        pl.semaphore_wait(barrier_sem, 2)

        sem_counter = [0]

        def rdma(src, dst, target):
            i = sem_counter[0]
            sem_counter[0] += 1
            return pltpu.make_async_remote_copy(
                src_ref=src, dst_ref=dst,
                send_sem=send_sems.at[i], recv_sem=recv_sems.at[i],
                device_id=(target,), device_id_type=pl.DeviceIdType.MESH,
            )

        def load_weights(slot, win_ref, wout_ref):
            winb_ref[slot] = win_ref[:, :].astype(BF16)
            woutb_ref[slot] = wout_ref[:, :].astype(BF16)

        def mlp(v_bf16, slot):
            h = jnp.dot(v_bf16, winb_ref[slot], preferred_element_type=F32)
            hb = jnp.maximum(h, 0.0).astype(BF16)
            return jnp.dot(hb, woutb_ref[slot],
                           preferred_element_type=F32).astype(BF16)

        def piece(w, c):
            return pl.ds(c * RW + 128 * w, RW)

        def half(w, h):
            return pl.ds(128 * w + HW * h, HW)

        all_d = []

        xb_ref[:, :] = x_ref[:, :].astype(BF16)
        ag = {}
        for w in (0, 1):
            ag[w] = []
            for k, p in enumerate(peers):
                d = rdma(xb_ref.at[pl.ds(RW * w, RW)], agb_ref.at[3 * w + k],
                         p)
                d.start()
                ag[w].append(d)
        all_d += ag[0] + ag[1]
        load_weights(0, win0_ref, wout0_ref)
        for w in (0, 1):
            partb_ref[piece(w, my), :] = mlp(xb_ref[pl.ds(RW * w, RW), :], 0)
        for k, p in enumerate(peers):
            for w in (0, 1):
                ag[w][k].wait_recv()
                partb_ref[piece(w, p), :] = mlp(agb_ref[3 * w + k], 0)

        s1 = {}
        for w in (0, 1):
            da = rdma(partb_ref.at[half(w, 0)], arb_ref.at[4 * w + 0], y_p)
            da.start()
            db = rdma(partb_ref.at[half(w, 1)], arb_ref.at[4 * w + 1], x_p)
            db.start()
            s1[w] = (da, db)
        load_weights(1, win1_ref, wout1_ref)

        for bnd in (0, 1):
            a8 = 8 * bnd
            s2 = {}
            for w in (0, 1):
                da, db = s1[w]
                da.wait_recv()
                sab_ref[4 * bnd + 2 * w + 0] = (
                    partb_ref[half(w, 0), :] + arb_ref[a8 + 4 * w + 0])
                d2a = rdma(sab_ref.at[4 * bnd + 2 * w + 0],
                           arb_ref.at[a8 + 4 * w + 2], x_p)
                d2a.start()
                db.wait_recv()
                sab_ref[4 * bnd + 2 * w + 1] = (
                    partb_ref[half(w, 1), :] + arb_ref[a8 + 4 * w + 1])
                d2b = rdma(sab_ref.at[4 * bnd + 2 * w + 1],
                           arb_ref.at[a8 + 4 * w + 3], y_p)
                d2b.start()
                s2[w] = (d2a, d2b)
                all_d += [d2a, d2b]
            ns1 = {}
            for w in (0, 1):
                d2a, d2b = s2[w]
                d2a.wait_recv()
                s1[w][0].wait_send()
                partb_ref[half(w, 0), :] = mlp(
                    sab_ref[4 * bnd + 2 * w + 0] + arb_ref[a8 + 4 * w + 2],
                    bnd + 1)
                d2b.wait_recv()
                s1[w][1].wait_send()
                partb_ref[half(w, 1), :] = mlp(
                    sab_ref[4 * bnd + 2 * w + 1] + arb_ref[a8 + 4 * w + 3],
                    bnd + 1)
                if bnd == 0:
                    na = rdma(partb_ref.at[half(w, 0)],
                              arb_ref.at[8 + 4 * w + 0], y_p)
                    na.start()
                    nb = rdma(partb_ref.at[half(w, 1)],
                              arb_ref.at[8 + 4 * w + 1], x_p)
                    nb.start()
                    ns1[w] = (na, nb)
                else:
                    rs = []
                    for k, p in enumerate(peers):
                        d = rdma(partb_ref.at[piece(w, p)],
                                 rsb_ref.at[3 * w + k], p)
                        d.start()
                        rs.append(d)
                    ns1[w] = tuple(rs)
                    all_d += rs
            if bnd == 0:
                load_weights(2, win2_ref, wout2_ref)
            s1 = ns1

        for w in (0, 1):
            for d in s1[w]:
                d.wait_recv()
            out_ref[pl.ds(RW * w, RW), :] = (
                partb_ref[piece(w, my), :].astype(F32)
                + rsb_ref[3 * w + 0].astype(F32)
                + rsb_ref[3 * w + 1].astype(F32)
                + rsb_ref[3 * w + 2].astype(F32))

        for d in all_d:
            d.wait_send()

    return pl.pallas_call(
        body,
        out_shape=jax.ShapeDtypeStruct((B_SH, D), F32),
        in_specs=[pl.BlockSpec(memory_space=pltpu.VMEM)] * 7,
        out_specs=pl.BlockSpec(memory_space=pltpu.VMEM),
        scratch_shapes=[
            pltpu.VMEM((B, D), BF16),
            pltpu.VMEM((B_SH, D), BF16),
            pltpu.VMEM((6, RW, D), BF16),
            pltpu.VMEM((8, HW, D), BF16),
            pltpu.VMEM((16, HW, D), BF16),
            pltpu.VMEM((6, RW, D), BF16),
            pltpu.VMEM((3, D, H_SH), BF16),
            pltpu.VMEM((3, H_SH, D), BF16),
            pltpu.SemaphoreType.DMA((N_RDMA,)),
            pltpu.SemaphoreType.DMA((N_RDMA,)),
        ],
        compiler_params=pltpu.CompilerParams(collective_id=0),
    )(x, Win0, Wout0, Win1, Wout1, Win2, Wout2)
